# baseline (speedup 1.0000x reference)
import functools

import jax
import jax.numpy as jnp
from jax import lax
from jax.experimental import pallas as pl
from jax.experimental.pallas import tpu as pltpu
from jax.experimental.pallas import tpu_sc as plsc

_B, _L, _D, _V = 128, 512, 512, 42
_NC, _NS, _LANES = 2, 16, 16
_NW = _NC * _NS
_LPW = _L // _NW
_CPR = _D // _LANES
_UNROLL = 4


def _positional_encoding():
    even_i = jnp.arange(0, _D, 2).astype(jnp.float32)
    denominator = jnp.power(10000.0, even_i / _D)
    position = jnp.arange(_L, dtype=jnp.float32).reshape(_L, 1)
    even_pe = jnp.sin(position / denominator)
    odd_pe = jnp.cos(position / denominator)
    return jnp.stack([even_pe, odd_pe], axis=2).reshape(_L, _D)


def _make_sc_kernel():
    mesh = plsc.VectorSubcoreMesh(core_axis_name="c", subcore_axis_name="s")

    @functools.partial(
        pl.kernel,
        mesh=mesh,
        out_type=jax.ShapeDtypeStruct((_B * _L, _D), jnp.float32),
        scratch_types=[
            pltpu.VMEM((_LPW, _D), jnp.float32),
            pltpu.VMEM((_LPW,), jnp.int32),
            pltpu.VMEM((_LPW, _D), jnp.float32),
            pltpu.SemaphoreType.DMA,
        ],
    )
    def sc_kernel(x_hbm, table_hbm, pe_hbm, out_hbm, pe_v, idx_v, buf_v, sem):
        wid = lax.axis_index("s") * _NC + lax.axis_index("c")
        base = wid * _LPW
        pltpu.sync_copy(pe_hbm.at[pl.ds(base, _LPW), :], pe_v)

        def b_body(b, carry):
            pltpu.sync_copy(x_hbm.at[pl.ds(b * _L + base, _LPW)], idx_v)
            pltpu.async_copy(table_hbm.at[idx_v], buf_v, sem).wait()
            for r in range(_LPW):

                def c_body(c, _, r=r):
                    for u in range(_UNROLL):
                        sl = pl.ds((c * _UNROLL + u) * _LANES, _LANES)
                        buf_v[r, sl] = buf_v[r, sl] + pe_v[r, sl]
                    return 0

                lax.fori_loop(0, _CPR // _UNROLL, c_body, 0)
            pltpu.sync_copy(buf_v, out_hbm.at[pl.ds(b * _L + base, _LPW), :])
            return carry

        lax.fori_loop(0, _B, b_body, 0)

    return sc_kernel


_SC_KERNEL = _make_sc_kernel()


def kernel(x, table, start_token, end_token):
    pe = _positional_encoding()
    out = _SC_KERNEL(x.reshape(-1), table, pe)
    return out.reshape(_B, _L, _D)


# speedup vs baseline: 1.5591x; 1.5591x over previous
import functools

import jax
import jax.numpy as jnp
from jax import lax
from jax.experimental import pallas as pl
from jax.experimental.pallas import tpu as pltpu
from jax.experimental.pallas import tpu_sc as plsc

_B, _L, _D, _V = 128, 512, 512, 42
_NC, _NS, _LANES = 2, 16, 16
_NW = _NC * _NS
_LPW = _L // _NW
_G = 2
_GR = _G * _LPW
_NGROUPS = _B // _G
_RING = 4
_CPR = _D // _LANES
_UNROLL = 4


def _positional_encoding():
    even_i = jnp.arange(0, _D, 2).astype(jnp.float32)
    denominator = jnp.power(10000.0, even_i / _D)
    position = jnp.arange(_L, dtype=jnp.float32).reshape(_L, 1)
    even_pe = jnp.sin(position / denominator)
    odd_pe = jnp.cos(position / denominator)
    return jnp.stack([even_pe, odd_pe], axis=2).reshape(_L, _D)


def _make_sc_kernel():
    mesh = plsc.VectorSubcoreMesh(core_axis_name="c", subcore_axis_name="s")

    @functools.partial(
        pl.kernel,
        mesh=mesh,
        out_type=jax.ShapeDtypeStruct((_B * _L, _D), jnp.float32),
        scratch_types=[
            pltpu.VMEM((_LPW, _D), jnp.float32),
            pltpu.VMEM((_B * _LPW,), jnp.int32),
            [pltpu.VMEM((_GR, _D), jnp.float32) for _ in range(_RING)],
            [pltpu.SemaphoreType.DMA for _ in range(_RING)],
            [pltpu.SemaphoreType.DMA for _ in range(_RING)],
        ],
    )
    def sc_kernel(x_hbm, table_hbm, pe_hbm, out_hbm,
                  pe_v, idx_v, bufs, gsems, osems):
        wid = lax.axis_index("s") * _NC + lax.axis_index("c")
        base = wid * _LPW
        pltpu.sync_copy(pe_hbm.at[pl.ds(base, _LPW), :], pe_v)
        pltpu.sync_copy(x_hbm.at[pl.ds(wid * _B * _LPW, _B * _LPW)], idx_v)

        def _gather(g, j):
            return pltpu.make_async_copy(
                table_hbm.at[idx_v.at[pl.ds(g * _GR, _GR)]], bufs[j], gsems[j])

        def _store(g, j, i):
            return pltpu.make_async_copy(
                bufs[j].at[pl.ds(i * _LPW, _LPW), :],
                out_hbm.at[pl.ds((g * _G + i) * _L + base, _LPW), :],
                osems[j])

        for j in range(_RING - 1):
            _gather(j, j).start()

        def k_body(k, carry):
            for j in range(_RING):
                g = k * _RING + j
                _gather(g, j).wait()
                w = (j + _RING - 1) % _RING
                nxt = g + _RING - 1

                @pl.when(g >= 1)
                def _drain_prev():
                    for i in range(_G):
                        _store(g - 1, w, i).wait()

                @pl.when(nxt < _NGROUPS)
                def _issue_next():
                    _gather(nxt, w).start()

                def r_body(r, _, j=j):
                    pr = r % _LPW

                    def c_body(c, _):
                        for u in range(_UNROLL):
                            sl = pl.ds((c * _UNROLL + u) * _LANES, _LANES)
                            bufs[j][r, sl] = bufs[j][r, sl] + pe_v[pr, sl]
                        return 0

                    lax.fori_loop(0, _CPR // _UNROLL, c_body, 0)
                    return 0

                lax.fori_loop(0, _GR, r_body, 0)
                for i in range(_G):
                    _store(g, j, i).start()
            return carry

        lax.fori_loop(0, _NGROUPS // _RING, k_body, 0)
        for i in range(_G):
            _store(_NGROUPS - 1, (_NGROUPS - 1) % _RING, i).wait()

    return sc_kernel


_SC_KERNEL = _make_sc_kernel()


def kernel(x, table, start_token, end_token):
    pe = _positional_encoding()
    xr = x.reshape(_B, _NW, _LPW).transpose(1, 0, 2).reshape(-1)
    out = _SC_KERNEL(xr, table, pe)
    return out.reshape(_B, _L, _D)
